# Initial kernel scaffold; baseline (speedup 1.0000x reference)
import jax
import jax.numpy as jnp
from jax.experimental import pallas as pl


def kernel(z, edge_index):
    raise NotImplementedError("write your pallas kernel here")


# speedup vs baseline: 1.2167x; 1.2167x over previous
import dataclasses
import functools

import jax
import jax.numpy as jnp
from jax import lax
from jax.experimental import pallas as pl
from jax.experimental.pallas import tpu as pltpu
from jax.experimental.pallas import tpu_sc as plsc

NC = 2
NS = 16
L = 16
NW = NC * NS

D = 256
W = 64
CHUNKS = 80
EPW = W * CHUNKS
E_PAD = EPW * NW


def _build_sc_call():
    mesh = plsc.VectorSubcoreMesh(core_axis_name="c", subcore_axis_name="s")
    cp = pltpu.CompilerParams()
    if "needs_layout_passes" in pltpu.CompilerParams.__dataclass_fields__:
        cp = dataclasses.replace(cp, needs_layout_passes=False)

    @functools.partial(
        pl.kernel,
        out_type=jax.ShapeDtypeStruct((E_PAD,), jnp.float32),
        mesh=mesh,
        scratch_types=[
            pltpu.VMEM((W,), jnp.int32),
            pltpu.VMEM((W,), jnp.int32),
            pltpu.VMEM((W, D), jnp.float32),
            pltpu.VMEM((W, D), jnp.float32),
            pltpu.VMEM((W,), jnp.float32),
            pltpu.VMEM((L, L), jnp.float32),
            pltpu.SemaphoreType.DMA,
        ],
        compiler_params=cp,
    )
    def sc_decode(z_hbm, src_hbm, dst_hbm, out_hbm, si, di, sr, dr, ob, mat, sem):
        wid = lax.axis_index("s") * NC + lax.axis_index("c")
        base = wid * EPW

        @pl.loop(0, CHUNKS)
        def _(c):
            off = base + c * W
            pltpu.sync_copy(src_hbm.at[pl.ds(off, W)], si)
            pltpu.sync_copy(dst_hbm.at[pl.ds(off, W)], di)
            cp_s = pltpu.async_copy(z_hbm.at[si], sr, sem)
            cp_d = pltpu.async_copy(z_hbm.at[di], dr, sem)
            cp_s.wait()
            cp_d.wait()

            rows16 = jnp.arange(L, dtype=jnp.int32)

            @pl.loop(0, W, step=L)
            def _(g):
                @pl.loop(0, L)
                def _(i):
                    e = g + i
                    acc = sr[e, pl.ds(0, L)] * dr[e, pl.ds(0, L)]
                    for j in range(1, D // L):
                        acc += sr[e, pl.ds(j * L, L)] * dr[e, pl.ds(j * L, L)]
                    mat[i, :] = acc

                tot = plsc.load_gather(mat, [rows16, jnp.zeros((L,), jnp.int32)])
                for l in range(1, L):
                    tot += plsc.load_gather(
                        mat, [rows16, jnp.full((L,), l, jnp.int32)]
                    )
                ob[pl.ds(g, L)] = 1.0 / (1.0 + jnp.exp(-tot))

            pltpu.sync_copy(ob, out_hbm.at[pl.ds(off, W)])

    return sc_decode


_SC_DECODE = _build_sc_call()


def kernel(z, edge_index):
    e = edge_index.shape[1]
    src = edge_index[0].astype(jnp.int32)
    dst = edge_index[1].astype(jnp.int32)
    pad = E_PAD - e
    src = jnp.concatenate([src, jnp.zeros((pad,), jnp.int32)])
    dst = jnp.concatenate([dst, jnp.zeros((pad,), jnp.int32)])
    out = _SC_DECODE(z, src, dst)
    return out[:e]


# speedup vs baseline: 1.6573x; 1.3622x over previous
import dataclasses
import functools

import jax
import jax.numpy as jnp
from jax import lax
from jax.experimental import pallas as pl
from jax.experimental.pallas import tpu as pltpu
from jax.experimental.pallas import tpu_sc as plsc

NC = 2
NS = 16
L = 16
NW = NC * NS

D = 256
W = 64
CHUNKS = 80
EPW = W * CHUNKS
E_PAD = EPW * NW


def _build_sc_call():
    mesh = plsc.VectorSubcoreMesh(core_axis_name="c", subcore_axis_name="s")
    cp = pltpu.CompilerParams()
    if "needs_layout_passes" in pltpu.CompilerParams.__dataclass_fields__:
        cp = dataclasses.replace(cp, needs_layout_passes=False)

    @functools.partial(
        pl.kernel,
        out_type=jax.ShapeDtypeStruct((E_PAD,), jnp.float32),
        mesh=mesh,
        scratch_types=[
            pltpu.VMEM((EPW,), jnp.int32),
            pltpu.VMEM((EPW,), jnp.int32),
            pltpu.VMEM((2 * W, D), jnp.float32),
            pltpu.VMEM((2 * W, D), jnp.float32),
            pltpu.VMEM((EPW,), jnp.float32),
            pltpu.VMEM((L, L), jnp.float32),
            pltpu.SemaphoreType.DMA,
            pltpu.SemaphoreType.DMA,
        ],
        compiler_params=cp,
    )
    def sc_decode(z_hbm, src_hbm, dst_hbm, out_hbm, si, di, sr, dr, ob, mat,
                  sem0, sem1):
        wid = lax.axis_index("s") * NC + lax.axis_index("c")
        base = wid * EPW
        sems = (sem0, sem1)
        rows16 = jnp.arange(L, dtype=jnp.int32)

        pltpu.sync_copy(src_hbm.at[pl.ds(base, EPW)], si)
        pltpu.sync_copy(dst_hbm.at[pl.ds(base, EPW)], di)

        def issue(c, b):
            dst_s = sr.at[pl.ds(b * W, W)]
            dst_d = dr.at[pl.ds(b * W, W)]
            pltpu.async_copy(z_hbm.at[si.at[pl.ds(c * W, W)]], dst_s, sems[b])
            pltpu.async_copy(z_hbm.at[di.at[pl.ds(c * W, W)]], dst_d, sems[b])

        def wait(b):
            pltpu.make_async_copy(
                z_hbm.at[si.at[pl.ds(0, W)]], sr.at[pl.ds(b * W, W)], sems[b]
            ).wait()
            pltpu.make_async_copy(
                z_hbm.at[di.at[pl.ds(0, W)]], dr.at[pl.ds(b * W, W)], sems[b]
            ).wait()

        def compute(c, b):
            @pl.loop(0, W, step=L)
            def _(g):
                @pl.loop(0, L)
                def _(i):
                    e = b * W + g + i
                    acc = sr[e, pl.ds(0, L)] * dr[e, pl.ds(0, L)]
                    for j in range(1, D // L):
                        acc += sr[e, pl.ds(j * L, L)] * dr[e, pl.ds(j * L, L)]
                    mat[i, :] = acc

                tot = plsc.load_gather(mat, [rows16, jnp.zeros((L,), jnp.int32)])
                for l in range(1, L):
                    tot += plsc.load_gather(
                        mat, [rows16, jnp.full((L,), l, jnp.int32)]
                    )
                ob[pl.ds(c * W + g, L)] = 1.0 / (1.0 + jnp.exp(-tot))

        issue(0, 0)

        @pl.loop(0, CHUNKS, step=2)
        def _(c):
            issue(c + 1, 1)
            wait(0)
            compute(c, 0)

            @pl.when(c + 2 < CHUNKS)
            def _():
                issue(c + 2, 0)

            wait(1)
            compute(c + 1, 1)

        pltpu.sync_copy(ob, out_hbm.at[pl.ds(base, EPW)])

    return sc_decode


_SC_DECODE = _build_sc_call()


def kernel(z, edge_index):
    e = edge_index.shape[1]
    src = edge_index[0].astype(jnp.int32)
    dst = edge_index[1].astype(jnp.int32)
    pad = E_PAD - e
    src = jnp.concatenate([src, jnp.zeros((pad,), jnp.int32)])
    dst = jnp.concatenate([dst, jnp.zeros((pad,), jnp.int32)])
    out = _SC_DECODE(z, src, dst)
    return out[:e]


# speedup vs baseline: 1.7745x; 1.0707x over previous
import dataclasses
import functools

import jax
import jax.numpy as jnp
from jax import lax
from jax.experimental import pallas as pl
from jax.experimental.pallas import tpu as pltpu
from jax.experimental.pallas import tpu_sc as plsc

NC = 2
NS = 16
L = 16
NW = NC * NS

D = 256
W = 64
CHUNKS = 80
EPW = W * CHUNKS
E_PAD = EPW * NW


def _build_sc_call():
    mesh = plsc.VectorSubcoreMesh(core_axis_name="c", subcore_axis_name="s")
    cp = pltpu.CompilerParams()
    if "needs_layout_passes" in pltpu.CompilerParams.__dataclass_fields__:
        cp = dataclasses.replace(cp, needs_layout_passes=False)

    @functools.partial(
        pl.kernel,
        out_type=jax.ShapeDtypeStruct((E_PAD,), jnp.float32),
        mesh=mesh,
        scratch_types=[
            pltpu.VMEM((EPW,), jnp.int32),
            pltpu.VMEM((EPW,), jnp.int32),
            pltpu.VMEM((2 * W, D // 2), jnp.int32),
            pltpu.VMEM((2 * W, D // 2), jnp.int32),
            pltpu.VMEM((EPW,), jnp.float32),
            pltpu.VMEM((L, L), jnp.float32),
            pltpu.SemaphoreType.DMA,
            pltpu.SemaphoreType.DMA,
        ],
        compiler_params=cp,
    )
    def sc_decode(z_hbm, src_hbm, dst_hbm, out_hbm, si, di, sr, dr, ob, mat,
                  sem0, sem1):
        wid = lax.axis_index("s") * NC + lax.axis_index("c")
        base = wid * EPW
        sems = (sem0, sem1)
        rows16 = jnp.arange(L, dtype=jnp.int32)

        pltpu.sync_copy(src_hbm.at[pl.ds(base, EPW)], si)
        pltpu.sync_copy(dst_hbm.at[pl.ds(base, EPW)], di)

        def issue(c, b):
            dst_s = sr.at[pl.ds(b * W, W)]
            dst_d = dr.at[pl.ds(b * W, W)]
            pltpu.async_copy(z_hbm.at[si.at[pl.ds(c * W, W)]], dst_s, sems[b])
            pltpu.async_copy(z_hbm.at[di.at[pl.ds(c * W, W)]], dst_d, sems[b])

        def wait(b):
            pltpu.make_async_copy(
                z_hbm.at[si.at[pl.ds(0, W)]], sr.at[pl.ds(b * W, W)], sems[b]
            ).wait()
            pltpu.make_async_copy(
                z_hbm.at[di.at[pl.ds(0, W)]], dr.at[pl.ds(b * W, W)], sems[b]
            ).wait()

        def compute(c, b):
            @pl.loop(0, W, step=L)
            def _(g):
                @pl.loop(0, L)
                def _(i):
                    e = b * W + g + i
                    acc0 = jnp.zeros((L,), jnp.float32)
                    acc1 = jnp.zeros((L,), jnp.float32)
                    for j in range(D // (2 * L)):
                        a = plsc.bitcast(sr[e, pl.ds(j * L, L)], jnp.bfloat16)
                        bb = plsc.bitcast(dr[e, pl.ds(j * L, L)], jnp.bfloat16)
                        a0, a1 = plsc.unpack(
                            a, format=plsc.PackFormat.INTERLEAVED,
                            preferred_element_type=jnp.float32)
                        b0, b1 = plsc.unpack(
                            bb, format=plsc.PackFormat.INTERLEAVED,
                            preferred_element_type=jnp.float32)
                        acc0 += a0 * b0
                        acc1 += a1 * b1
                    mat[i, :] = acc0 + acc1

                tot = plsc.load_gather(mat, [rows16, jnp.zeros((L,), jnp.int32)])
                for l in range(1, L):
                    tot += plsc.load_gather(
                        mat, [rows16, jnp.full((L,), l, jnp.int32)]
                    )
                ob[pl.ds(c * W + g, L)] = 1.0 / (1.0 + jnp.exp(-tot))

        issue(0, 0)

        @pl.loop(0, CHUNKS, step=2)
        def _(c):
            issue(c + 1, 1)
            wait(0)
            compute(c, 0)

            @pl.when(c + 2 < CHUNKS)
            def _():
                issue(c + 2, 0)

            wait(1)
            compute(c + 1, 1)

        pltpu.sync_copy(ob, out_hbm.at[pl.ds(base, EPW)])

    return sc_decode


_SC_DECODE = _build_sc_call()


def kernel(z, edge_index):
    e = edge_index.shape[1]
    src = edge_index[0].astype(jnp.int32)
    dst = edge_index[1].astype(jnp.int32)
    pad = E_PAD - e
    src = jnp.concatenate([src, jnp.zeros((pad,), jnp.int32)])
    dst = jnp.concatenate([dst, jnp.zeros((pad,), jnp.int32)])
    z_pairs = lax.bitcast_convert_type(
        z.astype(jnp.bfloat16).reshape(z.shape[0], z.shape[1] // 2, 2),
        jnp.int32)
    out = _SC_DECODE(z_pairs, src, dst)
    return out[:e]


# speedup vs baseline: 4.7602x; 2.6826x over previous
import dataclasses
import functools

import jax
import jax.numpy as jnp
from jax import lax
from jax.experimental import pallas as pl
from jax.experimental.pallas import tpu as pltpu
from jax.experimental.pallas import tpu_sc as plsc

NC = 2
NS = 16
L = 16
NW = NC * NS

D = 256
W = 64
CHUNKS = 80
EPW = W * CHUNKS
E_PAD = EPW * NW


def _build_sc_call():
    mesh = plsc.VectorSubcoreMesh(core_axis_name="c", subcore_axis_name="s")
    cp = pltpu.CompilerParams()
    if "needs_layout_passes" in pltpu.CompilerParams.__dataclass_fields__:
        cp = dataclasses.replace(cp, needs_layout_passes=False)

    @functools.partial(
        pl.kernel,
        out_type=jax.ShapeDtypeStruct((E_PAD,), jnp.float32),
        mesh=mesh,
        scratch_types=[
            pltpu.VMEM((EPW,), jnp.int32),
            pltpu.VMEM((EPW,), jnp.int32),
            pltpu.VMEM((2 * W, D // 2), jnp.int32),
            pltpu.VMEM((2 * W, D // 2), jnp.int32),
            pltpu.VMEM((EPW,), jnp.float32),
            pltpu.VMEM((L, L), jnp.float32),
            pltpu.VMEM_SHARED((10112, D // 2), jnp.int32),
            pltpu.SemaphoreType.DMA,
            pltpu.SemaphoreType.DMA,
        ],
        compiler_params=cp,
    )
    def sc_decode(z_hbm, src_hbm, dst_hbm, out_hbm, si, di, sr, dr, ob, mat,
                  zs, sem0, sem1):
        sid = lax.axis_index("s")
        wid = sid * NC + lax.axis_index("c")
        base = wid * EPW
        sems = (sem0, sem1)
        rows16 = jnp.arange(L, dtype=jnp.int32)

        nrows = 10112 // NS
        pltpu.sync_copy(z_hbm.at[pl.ds(sid * nrows, nrows)],
                        zs.at[pl.ds(sid * nrows, nrows)])
        pltpu.sync_copy(src_hbm.at[pl.ds(base, EPW)], si)
        pltpu.sync_copy(dst_hbm.at[pl.ds(base, EPW)], di)
        plsc.subcore_barrier()

        def issue(c, b):
            dst_s = sr.at[pl.ds(b * W, W)]
            dst_d = dr.at[pl.ds(b * W, W)]
            pltpu.async_copy(zs.at[si.at[pl.ds(c * W, W)]], dst_s, sems[b])
            pltpu.async_copy(zs.at[di.at[pl.ds(c * W, W)]], dst_d, sems[b])

        def wait(b):
            pltpu.make_async_copy(
                zs.at[si.at[pl.ds(0, W)]], sr.at[pl.ds(b * W, W)], sems[b]
            ).wait()
            pltpu.make_async_copy(
                zs.at[di.at[pl.ds(0, W)]], dr.at[pl.ds(b * W, W)], sems[b]
            ).wait()

        def compute(c, b):
            @pl.loop(0, W, step=L)
            def _(g):
                @pl.loop(0, L)
                def _(i):
                    e = b * W + g + i
                    acc0 = jnp.zeros((L,), jnp.float32)
                    acc1 = jnp.zeros((L,), jnp.float32)
                    for j in range(D // (2 * L)):
                        a = plsc.bitcast(sr[e, pl.ds(j * L, L)], jnp.bfloat16)
                        bb = plsc.bitcast(dr[e, pl.ds(j * L, L)], jnp.bfloat16)
                        a0, a1 = plsc.unpack(
                            a, format=plsc.PackFormat.INTERLEAVED,
                            preferred_element_type=jnp.float32)
                        b0, b1 = plsc.unpack(
                            bb, format=plsc.PackFormat.INTERLEAVED,
                            preferred_element_type=jnp.float32)
                        acc0 += a0 * b0
                        acc1 += a1 * b1
                    mat[i, :] = acc0 + acc1

                tot = plsc.load_gather(mat, [rows16, jnp.zeros((L,), jnp.int32)])
                for l in range(1, L):
                    tot += plsc.load_gather(
                        mat, [rows16, jnp.full((L,), l, jnp.int32)]
                    )
                ob[pl.ds(c * W + g, L)] = 1.0 / (1.0 + jnp.exp(-tot))

        issue(0, 0)

        @pl.loop(0, CHUNKS, step=2)
        def _(c):
            issue(c + 1, 1)
            wait(0)
            compute(c, 0)

            @pl.when(c + 2 < CHUNKS)
            def _():
                issue(c + 2, 0)

            wait(1)
            compute(c + 1, 1)

        pltpu.sync_copy(ob, out_hbm.at[pl.ds(base, EPW)])

    return sc_decode


_SC_DECODE = _build_sc_call()


def kernel(z, edge_index):
    e = edge_index.shape[1]
    src = edge_index[0].astype(jnp.int32)
    dst = edge_index[1].astype(jnp.int32)
    pad = E_PAD - e
    src = jnp.concatenate([src, jnp.zeros((pad,), jnp.int32)])
    dst = jnp.concatenate([dst, jnp.zeros((pad,), jnp.int32)])
    half = z.shape[1] // 2
    z16 = z.astype(jnp.bfloat16)
    lo = lax.bitcast_convert_type(z16[:, :half], jnp.uint16).astype(jnp.uint32)
    hi = lax.bitcast_convert_type(z16[:, half:], jnp.uint16).astype(jnp.uint32)
    z_pairs = lax.bitcast_convert_type(lo | (hi << jnp.uint32(16)), jnp.int32)
    z_pairs = jnp.pad(z_pairs, ((0, 10112 - z_pairs.shape[0]), (0, 0)))
    out = _SC_DECODE(z_pairs, src, dst)
    return out[:e]


# speedup vs baseline: 4.7676x; 1.0016x over previous
import dataclasses
import functools

import jax
import jax.numpy as jnp
from jax import lax
from jax.experimental import pallas as pl
from jax.experimental.pallas import tpu as pltpu
from jax.experimental.pallas import tpu_sc as plsc

NC = 2
NS = 16
L = 16
NW = NC * NS

D = 256
W = 64
CHUNKS = 80
EPW = W * CHUNKS
E_PAD = EPW * NW


def _build_sc_call():
    mesh = plsc.VectorSubcoreMesh(core_axis_name="c", subcore_axis_name="s")
    cp = pltpu.CompilerParams()
    if "needs_layout_passes" in pltpu.CompilerParams.__dataclass_fields__:
        cp = dataclasses.replace(cp, needs_layout_passes=False)

    @functools.partial(
        pl.kernel,
        out_type=jax.ShapeDtypeStruct((E_PAD,), jnp.float32),
        mesh=mesh,
        scratch_types=[
            pltpu.VMEM((EPW,), jnp.int32),
            pltpu.VMEM((EPW,), jnp.int32),
            pltpu.VMEM((2 * W, D // 2), jnp.int32),
            pltpu.VMEM((2 * W, D // 2), jnp.int32),
            pltpu.VMEM((EPW,), jnp.float32),
            pltpu.VMEM((L, L), jnp.float32),
            pltpu.VMEM_SHARED((10112, D // 2), jnp.int32),
            pltpu.SemaphoreType.DMA,
            pltpu.SemaphoreType.DMA,
        ],
        compiler_params=cp,
    )
    def sc_decode(z_hbm, src_hbm, dst_hbm, out_hbm, si, di, sr, dr, ob, mat,
                  zs, sem0, sem1):
        sid = lax.axis_index("s")
        wid = sid * NC + lax.axis_index("c")
        base = wid * EPW
        sems = (sem0, sem1)
        rows16 = jnp.arange(L, dtype=jnp.int32)

        nrows = 10112 // NS
        pltpu.sync_copy(z_hbm.at[pl.ds(sid * nrows, nrows)],
                        zs.at[pl.ds(sid * nrows, nrows)])
        pltpu.sync_copy(src_hbm.at[pl.ds(base, EPW)], si)
        pltpu.sync_copy(dst_hbm.at[pl.ds(base, EPW)], di)
        plsc.subcore_barrier()

        def issue(c, b):
            dst_s = sr.at[pl.ds(b * W, W)]
            dst_d = dr.at[pl.ds(b * W, W)]
            pltpu.async_copy(zs.at[si.at[pl.ds(c * W, W)]], dst_s, sems[b])
            pltpu.async_copy(zs.at[di.at[pl.ds(c * W, W)]], dst_d, sems[b])

        def wait(b):
            pltpu.make_async_copy(
                zs.at[si.at[pl.ds(0, W)]], sr.at[pl.ds(b * W, W)], sems[b]
            ).wait()
            pltpu.make_async_copy(
                zs.at[di.at[pl.ds(0, W)]], dr.at[pl.ds(b * W, W)], sems[b]
            ).wait()

        def compute(c, b):
            @pl.loop(0, W, step=L)
            def _(g):
                @pl.loop(0, L)
                def _(i):
                    e = b * W + g + i
                    acc0 = jnp.zeros((L,), jnp.float32)
                    acc1 = jnp.zeros((L,), jnp.float32)
                    mask = jnp.full((L,), -65536, jnp.int32)
                    for j in range(D // (2 * L)):
                        ws = sr[e, pl.ds(j * L, L)]
                        wd = dr[e, pl.ds(j * L, L)]
                        s_hi = plsc.bitcast(ws & mask, jnp.float32)
                        s_lo = plsc.bitcast(ws << 16, jnp.float32)
                        d_hi = plsc.bitcast(wd & mask, jnp.float32)
                        d_lo = plsc.bitcast(wd << 16, jnp.float32)
                        acc0 += s_hi * d_hi
                        acc1 += s_lo * d_lo
                    mat[i, :] = acc0 + acc1

                tot = plsc.load_gather(mat, [rows16, jnp.zeros((L,), jnp.int32)])
                for l in range(1, L):
                    tot += plsc.load_gather(
                        mat, [rows16, jnp.full((L,), l, jnp.int32)]
                    )
                ob[pl.ds(c * W + g, L)] = 1.0 / (1.0 + jnp.exp(-tot))

        issue(0, 0)

        @pl.loop(0, CHUNKS, step=2)
        def _(c):
            issue(c + 1, 1)
            wait(0)
            compute(c, 0)

            @pl.when(c + 2 < CHUNKS)
            def _():
                issue(c + 2, 0)

            wait(1)
            compute(c + 1, 1)

        pltpu.sync_copy(ob, out_hbm.at[pl.ds(base, EPW)])

    return sc_decode


_SC_DECODE = _build_sc_call()


def kernel(z, edge_index):
    e = edge_index.shape[1]
    src = edge_index[0].astype(jnp.int32)
    dst = edge_index[1].astype(jnp.int32)
    pad = E_PAD - e
    src = jnp.concatenate([src, jnp.zeros((pad,), jnp.int32)])
    dst = jnp.concatenate([dst, jnp.zeros((pad,), jnp.int32)])
    half = z.shape[1] // 2
    z16 = z.astype(jnp.bfloat16)
    lo = lax.bitcast_convert_type(z16[:, :half], jnp.uint16).astype(jnp.uint32)
    hi = lax.bitcast_convert_type(z16[:, half:], jnp.uint16).astype(jnp.uint32)
    z_pairs = lax.bitcast_convert_type(lo | (hi << jnp.uint32(16)), jnp.int32)
    z_pairs = jnp.pad(z_pairs, ((0, 10112 - z_pairs.shape[0]), (0, 0)))
    out = _SC_DECODE(z_pairs, src, dst)
    return out[:e]


# speedup vs baseline: 4.8498x; 1.0172x over previous
import dataclasses
import functools

import jax
import jax.numpy as jnp
from jax import lax
from jax.experimental import pallas as pl
from jax.experimental.pallas import tpu as pltpu
from jax.experimental.pallas import tpu_sc as plsc

NC = 2
NS = 16
L = 16
NW = NC * NS

D = 256
W = 64
CHUNKS = 80
EPW = W * CHUNKS
E_PAD = EPW * NW


def _build_sc_call():
    mesh = plsc.VectorSubcoreMesh(core_axis_name="c", subcore_axis_name="s")
    cp = pltpu.CompilerParams()
    if "needs_layout_passes" in pltpu.CompilerParams.__dataclass_fields__:
        cp = dataclasses.replace(cp, needs_layout_passes=False)

    @functools.partial(
        pl.kernel,
        out_type=jax.ShapeDtypeStruct((E_PAD,), jnp.float32),
        mesh=mesh,
        scratch_types=[
            pltpu.VMEM((EPW,), jnp.int32),
            pltpu.VMEM((EPW,), jnp.int32),
            pltpu.VMEM((2 * W, D // 2), jnp.int32),
            pltpu.VMEM((2 * W, D // 2), jnp.int32),
            pltpu.VMEM((EPW,), jnp.float32),
            pltpu.VMEM((L, L), jnp.float32),
            pltpu.VMEM_SHARED((10112, D // 2), jnp.int32),
            pltpu.SemaphoreType.DMA,
            pltpu.SemaphoreType.DMA,
        ],
        compiler_params=cp,
    )
    def sc_decode(z_hbm, src_hbm, dst_hbm, out_hbm, si, di, sr, dr, ob, mat,
                  zs, sem0, sem1):
        sid = lax.axis_index("s")
        wid = sid * NC + lax.axis_index("c")
        base = wid * EPW
        sems = (sem0, sem1)
        rows16 = jnp.arange(L, dtype=jnp.int32)

        nrows = 10112 // NS
        pltpu.sync_copy(z_hbm.at[pl.ds(sid * nrows, nrows)],
                        zs.at[pl.ds(sid * nrows, nrows)])
        pltpu.sync_copy(src_hbm.at[pl.ds(base, EPW)], si)
        pltpu.sync_copy(dst_hbm.at[pl.ds(base, EPW)], di)
        plsc.subcore_barrier()

        def issue(c, b):
            dst_s = sr.at[pl.ds(b * W, W)]
            dst_d = dr.at[pl.ds(b * W, W)]
            pltpu.async_copy(zs.at[si.at[pl.ds(c * W, W)]], dst_s, sems[b])
            pltpu.async_copy(zs.at[di.at[pl.ds(c * W, W)]], dst_d, sems[b])

        def wait(b):
            pltpu.make_async_copy(
                zs.at[si.at[pl.ds(0, W)]], sr.at[pl.ds(b * W, W)], sems[b]
            ).wait()
            pltpu.make_async_copy(
                zs.at[di.at[pl.ds(0, W)]], dr.at[pl.ds(b * W, W)], sems[b]
            ).wait()

        def compute(c, b):
            @pl.loop(0, W, step=L)
            def _(g):
                mask = jnp.full((L,), -65536, jnp.int32)
                for i in range(L):
                    e = b * W + g + i
                    acc0 = jnp.zeros((L,), jnp.float32)
                    acc1 = jnp.zeros((L,), jnp.float32)
                    for j in range(D // (2 * L)):
                        ws = sr[e, pl.ds(j * L, L)]
                        wd = dr[e, pl.ds(j * L, L)]
                        s_hi = plsc.bitcast(ws & mask, jnp.float32)
                        s_lo = plsc.bitcast(ws << 16, jnp.float32)
                        d_hi = plsc.bitcast(wd & mask, jnp.float32)
                        d_lo = plsc.bitcast(wd << 16, jnp.float32)
                        acc0 += s_hi * d_hi
                        acc1 += s_lo * d_lo
                    mat[i, :] = acc0 + acc1

                tot = plsc.load_gather(mat, [rows16, jnp.zeros((L,), jnp.int32)])
                for l in range(1, L):
                    tot += plsc.load_gather(
                        mat, [rows16, jnp.full((L,), l, jnp.int32)]
                    )
                ob[pl.ds(c * W + g, L)] = 1.0 / (1.0 + jnp.exp(-tot))

        issue(0, 0)

        @pl.loop(0, CHUNKS, step=2)
        def _(c):
            issue(c + 1, 1)
            wait(0)
            compute(c, 0)

            @pl.when(c + 2 < CHUNKS)
            def _():
                issue(c + 2, 0)

            wait(1)
            compute(c + 1, 1)

        pltpu.sync_copy(ob, out_hbm.at[pl.ds(base, EPW)])

    return sc_decode


_SC_DECODE = _build_sc_call()


def kernel(z, edge_index):
    e = edge_index.shape[1]
    src = edge_index[0].astype(jnp.int32)
    dst = edge_index[1].astype(jnp.int32)
    pad = E_PAD - e
    src = jnp.concatenate([src, jnp.zeros((pad,), jnp.int32)])
    dst = jnp.concatenate([dst, jnp.zeros((pad,), jnp.int32)])
    half = z.shape[1] // 2
    z16 = z.astype(jnp.bfloat16)
    lo = lax.bitcast_convert_type(z16[:, :half], jnp.uint16).astype(jnp.uint32)
    hi = lax.bitcast_convert_type(z16[:, half:], jnp.uint16).astype(jnp.uint32)
    z_pairs = lax.bitcast_convert_type(lo | (hi << jnp.uint32(16)), jnp.int32)
    z_pairs = jnp.pad(z_pairs, ((0, 10112 - z_pairs.shape[0]), (0, 0)))
    out = _SC_DECODE(z_pairs, src, dst)
    return out[:e]


# speedup vs baseline: 5.3652x; 1.1063x over previous
import dataclasses
import functools

import jax
import jax.numpy as jnp
from jax import lax
from jax.experimental import pallas as pl
from jax.experimental.pallas import tpu as pltpu
from jax.experimental.pallas import tpu_sc as plsc

NC = 2
NS = 16
L = 16
NW = NC * NS

D = 256
W = 64
CHUNKS = 80
EPW = W * CHUNKS
E_PAD = EPW * NW


def _build_sc_call():
    mesh = plsc.VectorSubcoreMesh(core_axis_name="c", subcore_axis_name="s")
    cp = pltpu.CompilerParams()
    if "needs_layout_passes" in pltpu.CompilerParams.__dataclass_fields__:
        cp = dataclasses.replace(cp, needs_layout_passes=False)

    @functools.partial(
        pl.kernel,
        out_type=jax.ShapeDtypeStruct((E_PAD,), jnp.float32),
        mesh=mesh,
        scratch_types=[
            pltpu.VMEM((EPW,), jnp.int32),
            pltpu.VMEM((EPW,), jnp.int32),
            pltpu.VMEM((2 * W, D // 2), jnp.int32),
            pltpu.VMEM((2 * W, D // 2), jnp.int32),
            pltpu.VMEM((EPW,), jnp.float32),
            pltpu.VMEM((L, L), jnp.float32),
            pltpu.VMEM_SHARED((10112, D // 2), jnp.int32),
            pltpu.SemaphoreType.DMA,
            pltpu.SemaphoreType.DMA,
        ],
        compiler_params=cp,
    )
    def sc_decode(z_hbm, src_hbm, dst_hbm, out_hbm, si, di, sr, dr, ob, mat,
                  zs, sem0, sem1):
        sid = lax.axis_index("s")
        wid = sid * NC + lax.axis_index("c")
        base = wid * EPW
        sems = (sem0, sem1)
        rows16 = jnp.arange(L, dtype=jnp.int32)

        nrows = 10112 // NS
        pltpu.sync_copy(z_hbm.at[pl.ds(sid * nrows, nrows)],
                        zs.at[pl.ds(sid * nrows, nrows)])
        pltpu.sync_copy(src_hbm.at[pl.ds(base, EPW)], si)
        pltpu.sync_copy(dst_hbm.at[pl.ds(base, EPW)], di)
        plsc.subcore_barrier()

        def issue(c, b):
            dst_s = sr.at[pl.ds(b * W, W)]
            dst_d = dr.at[pl.ds(b * W, W)]
            pltpu.async_copy(zs.at[si.at[pl.ds(c * W, W)]], dst_s, sems[b])
            pltpu.async_copy(zs.at[di.at[pl.ds(c * W, W)]], dst_d, sems[b])

        def wait(b):
            pltpu.make_async_copy(
                zs.at[si.at[pl.ds(0, W)]], sr.at[pl.ds(b * W, W)], sems[b]
            ).wait()
            pltpu.make_async_copy(
                zs.at[di.at[pl.ds(0, W)]], dr.at[pl.ds(b * W, W)], sems[b]
            ).wait()

        def compute(c, b):
            @pl.loop(0, W, step=L)
            def _(g):
                mask = jnp.full((L,), -65536, jnp.int32)
                for i in range(L):
                    e = b * W + g + i
                    acc0 = jnp.zeros((L,), jnp.float32)
                    acc1 = jnp.zeros((L,), jnp.float32)
                    for j in range(D // (2 * L)):
                        ws = sr[e, pl.ds(j * L, L)]
                        wd = dr[e, pl.ds(j * L, L)]
                        ps = plsc.bitcast(ws, jnp.bfloat16) * plsc.bitcast(
                            wd, jnp.bfloat16)
                        pw = plsc.bitcast(ps, jnp.int32)
                        acc0 += plsc.bitcast(pw & mask, jnp.float32)
                        acc1 += plsc.bitcast(pw << 16, jnp.float32)
                    mat[i, :] = acc0 + acc1

                tot = plsc.load_gather(mat, [rows16, jnp.zeros((L,), jnp.int32)])
                for l in range(1, L):
                    tot += plsc.load_gather(
                        mat, [rows16, jnp.full((L,), l, jnp.int32)]
                    )
                ob[pl.ds(c * W + g, L)] = 1.0 / (1.0 + jnp.exp(-tot))

        issue(0, 0)

        @pl.loop(0, CHUNKS, step=2)
        def _(c):
            issue(c + 1, 1)
            wait(0)
            compute(c, 0)

            @pl.when(c + 2 < CHUNKS)
            def _():
                issue(c + 2, 0)

            wait(1)
            compute(c + 1, 1)

        pltpu.sync_copy(ob, out_hbm.at[pl.ds(base, EPW)])

    return sc_decode


_SC_DECODE = _build_sc_call()


def kernel(z, edge_index):
    e = edge_index.shape[1]
    src = edge_index[0].astype(jnp.int32)
    dst = edge_index[1].astype(jnp.int32)
    pad = E_PAD - e
    src = jnp.concatenate([src, jnp.zeros((pad,), jnp.int32)])
    dst = jnp.concatenate([dst, jnp.zeros((pad,), jnp.int32)])
    half = z.shape[1] // 2
    z16 = z.astype(jnp.bfloat16)
    lo = lax.bitcast_convert_type(z16[:, :half], jnp.uint16).astype(jnp.uint32)
    hi = lax.bitcast_convert_type(z16[:, half:], jnp.uint16).astype(jnp.uint32)
    z_pairs = lax.bitcast_convert_type(lo | (hi << jnp.uint32(16)), jnp.int32)
    z_pairs = jnp.pad(z_pairs, ((0, 10112 - z_pairs.shape[0]), (0, 0)))
    out = _SC_DECODE(z_pairs, src, dst)
    return out[:e]
